# Initial kernel scaffold; baseline (speedup 1.0000x reference)
import jax
import jax.numpy as jnp
from jax.experimental import pallas as pl


def kernel(X1, X2):
    raise NotImplementedError("write your pallas kernel here")


# speedup vs baseline: 2.7447x; 2.7447x over previous
import functools

import jax
import jax.numpy as jnp
from jax.experimental import pallas as pl

B, N, D = 32, 32, 256
IB = 8
DH = 128


def _body(x1_ref, x2_ref, o_ref):
    x1 = x1_ref[0]
    x2 = x2_ref[0]
    for dh in range(D // DH):
        dsl = slice(dh * DH, (dh + 1) * DH)
        acc = jnp.zeros((IB, N, DH), jnp.float32)
        for k in range(N):
            a = x1[:, k, dsl]
            b = x2[k, :, dsl]
            acc = acc + a[:, None, :] * b[None, :, :]
        o_ref[0, :, :, dsl] = acc


@jax.jit
def kernel(X1, X2):
    return pl.pallas_call(
        _body,
        grid=(B, N // IB),
        in_specs=[
            pl.BlockSpec((1, IB, N, D), lambda b, ib: (b, ib, 0, 0)),
            pl.BlockSpec((1, N, N, D), lambda b, ib: (b, 0, 0, 0)),
        ],
        out_specs=pl.BlockSpec((1, IB, N, D), lambda b, ib: (b, ib, 0, 0)),
        out_shape=jax.ShapeDtypeStruct((B, N, N, D), jnp.float32),
    )(X1, X2)


# speedup vs baseline: 2.8259x; 1.0296x over previous
import functools

import jax
import jax.numpy as jnp
from jax.experimental import pallas as pl

B, N, D = 32, 32, 256
IB = 8
DH = 128


JH = 8


def _body(x1_ref, x2_ref, o_ref):
    for dh in range(D // DH):
        dsl = slice(dh * DH, (dh + 1) * DH)
        for jh in range(N // JH):
            jsl = slice(jh * JH, (jh + 1) * JH)
            acc = jnp.zeros((IB, JH, DH), jnp.float32)
            for k in range(N):
                a = x1_ref[0, :, k, dsl]
                b = x2_ref[0, k, jsl, dsl]
                acc = acc + a[:, None, :] * b[None, :, :]
            o_ref[0, :, jsl, dsl] = acc


@jax.jit
def kernel(X1, X2):
    return pl.pallas_call(
        _body,
        grid=(B, N // IB),
        in_specs=[
            pl.BlockSpec((1, IB, N, D), lambda b, ib: (b, ib, 0, 0)),
            pl.BlockSpec((1, N, N, D), lambda b, ib: (b, 0, 0, 0)),
        ],
        out_specs=pl.BlockSpec((1, IB, N, D), lambda b, ib: (b, ib, 0, 0)),
        out_shape=jax.ShapeDtypeStruct((B, N, N, D), jnp.float32),
    )(X1, X2)


# speedup vs baseline: 4.7700x; 1.6880x over previous
import jax
import jax.numpy as jnp
from jax.experimental import pallas as pl

B, N, D = 32, 32, 256
IB = 8
JH = 8
DH = 128


def _body(x1_ref, x2_ref, o_ref):
    for ib in range(N // IB):
        isl = slice(ib * IB, (ib + 1) * IB)
        for dh in range(D // DH):
            dsl = slice(dh * DH, (dh + 1) * DH)
            for jh in range(N // JH):
                jsl = slice(jh * JH, (jh + 1) * JH)
                acc = jnp.zeros((IB, JH, DH), jnp.float32)
                for k in range(N):
                    a = x1_ref[0, isl, k, dsl]
                    b = x2_ref[0, k, jsl, dsl]
                    acc = acc + a[:, None, :] * b[None, :, :]
                o_ref[0, isl, jsl, dsl] = acc


@jax.jit
def kernel(X1, X2):
    return pl.pallas_call(
        _body,
        grid=(B,),
        in_specs=[
            pl.BlockSpec((1, N, N, D), lambda b: (b, 0, 0, 0)),
            pl.BlockSpec((1, N, N, D), lambda b: (b, 0, 0, 0)),
        ],
        out_specs=pl.BlockSpec((1, N, N, D), lambda b: (b, 0, 0, 0)),
        out_shape=jax.ShapeDtypeStruct((B, N, N, D), jnp.float32),
    )(X1, X2)
